# baseline (speedup 1.0000x reference)
import functools

import jax
import jax.numpy as jnp
from jax import lax
from jax.experimental import pallas as pl
from jax.experimental.pallas import tpu as pltpu
from jax.experimental.pallas import tpu_sc as plsc

B = 16384
EMBED_DIM = 32
NUM_CATEGORIES = 1000000

NC = 2
NS = 16
NW = NC * NS
BPW = B // NW
IDX_CHUNK = 128
NCHUNK = BPW // IDX_CHUNK


def _sc_embed_body(x_hbm, w_hbm, out_hbm, xv, idxv, rows, sem):
    wid = lax.axis_index("c") * NS + lax.axis_index("s")
    base = wid * BPW
    pltpu.sync_copy(x_hbm.at[pl.ds(base, BPW)], xv)
    lanes = lax.iota(jnp.int32, 16)
    col0 = jnp.zeros((16,), jnp.int32)
    for i in range(BPW // 16):
        rowi = lanes + (i * 16)
        vals = plsc.load_gather(xv, [rowi, col0])
        ids = jnp.clip(vals.astype(jnp.int32), 0, NUM_CATEGORIES)
        idxv[i // (IDX_CHUNK // 16), pl.ds((i % (IDX_CHUNK // 16)) * 16, 16)] = ids
    copies = [
        pltpu.async_copy(
            w_hbm.at[idxv.at[j]],
            rows.at[pl.ds(j * IDX_CHUNK, IDX_CHUNK)],
            sem,
        )
        for j in range(NCHUNK)
    ]
    for c in copies:
        c.wait()
    pltpu.sync_copy(rows, out_hbm.at[pl.ds(base, BPW)])


_sc_embed = functools.partial(
    pl.kernel,
    out_type=jax.ShapeDtypeStruct((B, EMBED_DIM), jnp.float32),
    mesh=plsc.VectorSubcoreMesh(core_axis_name="c", subcore_axis_name="s"),
    compiler_params=pltpu.CompilerParams(
        needs_layout_passes=False, use_tc_tiling_on_sc=False
    ),
    scratch_types=[
        pltpu.VMEM((BPW, 4), jnp.float32),
        pltpu.VMEM((NCHUNK, IDX_CHUNK), jnp.int32),
        pltpu.VMEM((BPW, EMBED_DIM), jnp.float32),
        pltpu.SemaphoreType.DMA,
    ],
)(_sc_embed_body)


def _bn_body(x_ref, g_ref, b_ref, o_ref):
    xv = x_ref[...]
    m = jnp.mean(xv, axis=0, keepdims=True)
    d = xv - m
    v = jnp.mean(d * d, axis=0, keepdims=True)
    o_ref[...] = d * lax.rsqrt(v + 1e-5) * g_ref[...] + b_ref[...]


def _tc_batchnorm(x, g4, b4):
    return pl.pallas_call(
        _bn_body,
        out_shape=jax.ShapeDtypeStruct((B, 4), jnp.float32),
    )(x, g4, b4)


def kernel(x, W, gamma, beta):
    embed = _sc_embed(x, W)
    g4 = jnp.concatenate([jnp.ones((1,), jnp.float32), gamma]).reshape(1, 4)
    b4 = jnp.concatenate([jnp.zeros((1,), jnp.float32), beta]).reshape(1, 4)
    bn = _tc_batchnorm(x, g4, b4)
    return jnp.concatenate([embed, bn[:, 1:]], axis=1)


# speedup vs baseline: 4.5421x; 4.5421x over previous
import functools

import jax
import jax.numpy as jnp
from jax import lax
from jax.experimental import pallas as pl
from jax.experimental.pallas import tpu as pltpu
from jax.experimental.pallas import tpu_sc as plsc

B = 16384
EMBED_DIM = 32
NUM_CATEGORIES = 1000000
V = NUM_CATEGORIES + 1

NC = 2
NS = 16
NW = NC * NS
BPW = B // NW
NBUF = 8


def _sc_embed_body(xt_hbm, wt_hbm, out_hbm, idsf, idsi, wins, obuf, sem):
    wid = lax.axis_index("c") * NS + lax.axis_index("s")
    base = pl.multiple_of(wid * BPW, BPW)
    pltpu.sync_copy(xt_hbm.at[pl.ds(0, 1), pl.ds(base, BPW)], idsf)
    for i in range(BPW // 16):
        v = idsf[0, pl.ds(i * 16, 16)]
        idsi[0, pl.ds(i * 16, 16)] = jnp.clip(v.astype(jnp.int32), 0, NUM_CATEGORIES)
    lanes = lax.iota(jnp.int32, 16)
    r_lo = lanes
    r_hi = lanes + 16

    def id_at(j):
        vec = idsi[0, pl.ds(pl.multiple_of((j // 16) * 16, 16), 16)]
        return jnp.sum(jnp.where(lanes == (j % 16), vec, 0))

    def fetch(j, b):
        q = pl.multiple_of((id_at(j) // 128) * 128, 128)
        pltpu.async_copy(wt_hbm.at[:, pl.ds(q, 128)], wins.at[b], sem)

    def drain(b):
        pltpu.make_async_copy(wt_hbm.at[:, pl.ds(0, 128)], wins.at[b], sem).wait()

    for j in range(NBUF):
        fetch(j, j)

    def body(j, carry):
        b = lax.rem(j, NBUF)
        drain(b)
        idv = id_at(j)
        c16 = jnp.full((16,), lax.rem(idv, 128), jnp.int32)
        lo = plsc.load_gather(wins.at[b], [r_lo, c16])
        hi = plsc.load_gather(wins.at[b], [r_hi, c16])
        obuf[j, pl.ds(0, 16)] = lo
        obuf[j, pl.ds(16, 16)] = hi

        @pl.when(j < BPW - NBUF)
        def _():
            fetch(j + NBUF, b)

        return carry

    lax.fori_loop(0, BPW, body, 0)
    pltpu.sync_copy(obuf, out_hbm.at[pl.ds(base, BPW)])


_sc_embed = functools.partial(
    pl.kernel,
    out_type=jax.ShapeDtypeStruct((B, EMBED_DIM), jnp.float32),
    mesh=plsc.VectorSubcoreMesh(core_axis_name="c", subcore_axis_name="s"),
    compiler_params=pltpu.CompilerParams(needs_layout_passes=False),
    scratch_types=[
        pltpu.VMEM((1, BPW), jnp.float32),
        pltpu.VMEM((1, BPW), jnp.int32),
        pltpu.VMEM((NBUF, 32, 128), jnp.float32),
        pltpu.VMEM((BPW, EMBED_DIM), jnp.float32),
        pltpu.SemaphoreType.DMA,
    ],
)(_sc_embed_body)


def _bn_body(xt_ref, g_ref, b_ref, o_ref):
    xv = xt_ref[...]
    m = jnp.mean(xv, axis=1, keepdims=True)
    d = xv - m
    v = jnp.mean(d * d, axis=1, keepdims=True)
    o_ref[...] = d * lax.rsqrt(v + 1e-5) * g_ref[...] + b_ref[...]


def _tc_batchnorm(xt, g4, b4):
    return pl.pallas_call(
        _bn_body,
        out_shape=jax.ShapeDtypeStruct((4, B), jnp.float32),
    )(xt, g4, b4)


def kernel(x, W, gamma, beta):
    xt = x.T
    wt = W.T
    emb = _sc_embed(xt, wt)
    g4 = jnp.concatenate([jnp.ones((1,), jnp.float32), gamma]).reshape(4, 1)
    b4 = jnp.concatenate([jnp.zeros((1,), jnp.float32), beta]).reshape(4, 1)
    bnt = _tc_batchnorm(xt, g4, b4)
    return jnp.concatenate([emb, bnt[1:, :].T], axis=1)


# speedup vs baseline: 4.6852x; 1.0315x over previous
import functools

import jax
import jax.numpy as jnp
from jax import lax
from jax.experimental import pallas as pl
from jax.experimental.pallas import tpu as pltpu
from jax.experimental.pallas import tpu_sc as plsc

B = 16384
EMBED_DIM = 32
NUM_CATEGORIES = 1000000
V = NUM_CATEGORIES + 1

NC = 2
NS = 16
NW = NC * NS
BPW = B // NW
NBUF = 12


def _sc_embed_body(xt_hbm, wt_hbm, out_hbm, idsf, idsi, wins, obuf, sem):
    wid = lax.axis_index("c") * NS + lax.axis_index("s")
    base = pl.multiple_of(wid * BPW, BPW)
    pltpu.sync_copy(xt_hbm.at[pl.ds(0, 1), pl.ds(base, BPW)], idsf)
    for i in range(BPW // 16):
        v = idsf[0, pl.ds(i * 16, 16)]
        idsi[0, pl.ds(i * 16, 16)] = jnp.clip(v.astype(jnp.int32), 0, NUM_CATEGORIES)
    lanes = lax.iota(jnp.int32, 16)
    r_lo = lanes
    r_hi = lanes + 16

    def id_at(j):
        vec = idsi[0, pl.ds(pl.multiple_of((j // 16) * 16, 16), 16)]
        return jnp.sum(jnp.where(lanes == (j % 16), vec, 0))

    def fetch(j, b):
        q = pl.multiple_of((id_at(j) // 128) * 128, 128)
        pltpu.async_copy(wt_hbm.at[:, pl.ds(q, 128)], wins.at[b], sem)

    def drain(b):
        pltpu.make_async_copy(wt_hbm.at[:, pl.ds(0, 128)], wins.at[b], sem).wait()

    for j in range(NBUF):
        fetch(j, j)

    def body(j, carry):
        b = lax.rem(j, NBUF)
        drain(b)
        idv = id_at(j)
        c16 = jnp.full((16,), lax.rem(idv, 128), jnp.int32)
        lo = plsc.load_gather(wins.at[b], [r_lo, c16])
        hi = plsc.load_gather(wins.at[b], [r_hi, c16])
        obuf[j, pl.ds(0, 16)] = lo
        obuf[j, pl.ds(16, 16)] = hi

        @pl.when(j < BPW - NBUF)
        def _():
            fetch(j + NBUF, b)

        return carry

    lax.fori_loop(0, BPW, body, 0)
    pltpu.sync_copy(obuf, out_hbm.at[pl.ds(base, BPW)])


_sc_embed = functools.partial(
    pl.kernel,
    out_type=jax.ShapeDtypeStruct((B, EMBED_DIM), jnp.float32),
    mesh=plsc.VectorSubcoreMesh(core_axis_name="c", subcore_axis_name="s"),
    compiler_params=pltpu.CompilerParams(needs_layout_passes=False),
    scratch_types=[
        pltpu.VMEM((1, BPW), jnp.float32),
        pltpu.VMEM((1, BPW), jnp.int32),
        pltpu.VMEM((NBUF, 32, 128), jnp.float32),
        pltpu.VMEM((BPW, EMBED_DIM), jnp.float32),
        pltpu.SemaphoreType.DMA,
    ],
)(_sc_embed_body)


def _bn_body(xt_ref, g_ref, b_ref, o_ref):
    xv = xt_ref[...]
    m = jnp.mean(xv, axis=1, keepdims=True)
    d = xv - m
    v = jnp.mean(d * d, axis=1, keepdims=True)
    o_ref[...] = d * lax.rsqrt(v + 1e-5) * g_ref[...] + b_ref[...]


def _tc_batchnorm(xt, g4, b4):
    return pl.pallas_call(
        _bn_body,
        out_shape=jax.ShapeDtypeStruct((4, B), jnp.float32),
    )(xt, g4, b4)


def kernel(x, W, gamma, beta):
    xt = x.T
    wt = W.T
    emb = _sc_embed(xt, wt)
    g4 = jnp.concatenate([jnp.ones((1,), jnp.float32), gamma]).reshape(4, 1)
    b4 = jnp.concatenate([jnp.zeros((1,), jnp.float32), beta]).reshape(4, 1)
    bnt = _tc_batchnorm(xt, g4, b4)
    return jnp.concatenate([emb, bnt[1:, :].T], axis=1)


# speedup vs baseline: 4.6921x; 1.0015x over previous
import functools

import jax
import jax.numpy as jnp
from jax import lax
from jax.experimental import pallas as pl
from jax.experimental.pallas import tpu as pltpu
from jax.experimental.pallas import tpu_sc as plsc

B = 16384
EMBED_DIM = 32
NUM_CATEGORIES = 1000000
V = NUM_CATEGORIES + 1

NC = 2
NS = 16
NW = NC * NS
BPW = B // NW
NBUF = 12


def _sc_embed_body(xt_hbm, wt_hbm, out_hbm, idsf, idsi, wins, obuf, sem):
    wid = lax.axis_index("c") * NS + lax.axis_index("s")
    base = pl.multiple_of(wid * BPW, BPW)
    pltpu.sync_copy(xt_hbm.at[pl.ds(0, 1), pl.ds(base, BPW)], idsf)
    for i in range(BPW // 16):
        v = idsf[0, pl.ds(i * 16, 16)]
        idsi[0, pl.ds(i * 16, 16)] = jnp.clip(v.astype(jnp.int32), 0, NUM_CATEGORIES)
    lanes = lax.iota(jnp.int32, 16)
    r_lo = lanes
    r_hi = lanes + 16

    def id_at(j):
        vec = idsi[0, pl.ds(pl.multiple_of((j // 16) * 16, 16), 16)]
        return jnp.sum(jnp.where(lanes == (j % 16), vec, 0))

    def fetch(j, b):
        q = pl.multiple_of((id_at(j) // 128) * 128, 128)
        for t in range(4):
            pltpu.async_copy(
                wt_hbm.at[pl.ds(8 * t, 8), pl.ds(q, 128)],
                wins.at[b, pl.ds(8 * t, 8), :],
                sem,
            )

    def drain(b):
        pltpu.make_async_copy(wt_hbm.at[:, pl.ds(0, 128)], wins.at[b], sem).wait()

    for j in range(NBUF):
        fetch(j, j)

    def body(j, carry):
        b = lax.rem(j, NBUF)
        drain(b)
        idv = id_at(j)
        c16 = jnp.full((16,), lax.rem(idv, 128), jnp.int32)
        lo = plsc.load_gather(wins.at[b], [r_lo, c16])
        hi = plsc.load_gather(wins.at[b], [r_hi, c16])
        obuf[j, pl.ds(0, 16)] = lo
        obuf[j, pl.ds(16, 16)] = hi

        @pl.when(j < BPW - NBUF)
        def _():
            fetch(j + NBUF, b)

        return carry

    lax.fori_loop(0, BPW, body, 0)
    pltpu.sync_copy(obuf, out_hbm.at[pl.ds(base, BPW)])


_sc_embed = functools.partial(
    pl.kernel,
    out_type=jax.ShapeDtypeStruct((B, EMBED_DIM), jnp.float32),
    mesh=plsc.VectorSubcoreMesh(core_axis_name="c", subcore_axis_name="s"),
    compiler_params=pltpu.CompilerParams(needs_layout_passes=False),
    scratch_types=[
        pltpu.VMEM((1, BPW), jnp.float32),
        pltpu.VMEM((1, BPW), jnp.int32),
        pltpu.VMEM((NBUF, 32, 128), jnp.float32),
        pltpu.VMEM((BPW, EMBED_DIM), jnp.float32),
        pltpu.SemaphoreType.DMA,
    ],
)(_sc_embed_body)


def _bn_body(xt_ref, g_ref, b_ref, o_ref):
    xv = xt_ref[...]
    m = jnp.mean(xv, axis=1, keepdims=True)
    d = xv - m
    v = jnp.mean(d * d, axis=1, keepdims=True)
    o_ref[...] = d * lax.rsqrt(v + 1e-5) * g_ref[...] + b_ref[...]


def _tc_batchnorm(xt, g4, b4):
    return pl.pallas_call(
        _bn_body,
        out_shape=jax.ShapeDtypeStruct((4, B), jnp.float32),
    )(xt, g4, b4)


def kernel(x, W, gamma, beta):
    xt = x.T
    wt = W.T
    emb = _sc_embed(xt, wt)
    g4 = jnp.concatenate([jnp.ones((1,), jnp.float32), gamma]).reshape(4, 1)
    b4 = jnp.concatenate([jnp.zeros((1,), jnp.float32), beta]).reshape(4, 1)
    bnt = _tc_batchnorm(xt, g4, b4)
    return jnp.concatenate([emb, bnt[1:, :].T], axis=1)
